# baseline (speedup 1.0000x reference)
import functools

import jax
import jax.numpy as jnp
from jax import lax
from jax.experimental import pallas as pl
from jax.experimental.pallas import tpu as pltpu
from jax.experimental.pallas import tpu_sc as plsc

DIM = 1024
HID = 4096
NE = 8
TOPK = 2

BT = 512
BH = 2048
NJ = HID // BH
T_TOK = 4096
RPAD = TOPK * T_TOK + NE * BT
NT = RPAD // BT

_NEG_INF = float("-inf")
_INV_SQRT2 = 0.7071067811865476


def _gelu_exact(v):
    return 0.5 * v * (1.0 + jax.lax.erf(v * _INV_SQRT2))


def _router_kernel(x_ref, rw_ref, p0_ref, p1_ref, g0_ref, g1_ref, cnt_ref,
                   aux_ref):
    x = x_ref[...]
    rw = rw_ref[...]
    logits = jax.lax.dot_general(
        x, rw, (((1,), (1,)), ((), ())),
        preferred_element_type=jnp.float32)
    m = jnp.max(logits, axis=-1, keepdims=True)
    ex = jnp.exp(logits - m)
    probs = ex / jnp.sum(ex, axis=-1, keepdims=True)

    T = x.shape[0]
    iota_e = jax.lax.broadcasted_iota(jnp.int32, (T, NE), 1)
    a0 = jnp.argmax(probs, axis=-1)[:, None]
    oh0 = (iota_e == a0)
    m0 = jnp.max(probs, axis=-1, keepdims=True)
    masked = jnp.where(oh0, _NEG_INF, probs)
    a1 = jnp.argmax(masked, axis=-1)[:, None]
    oh1 = (iota_e == a1)
    m1 = jnp.max(masked, axis=-1, keepdims=True)
    s = m0 + m1
    g0_ref[...] = m0 / s
    g1_ref[...] = m1 / s

    p0_ref[...] = a0.astype(jnp.int32)
    p1_ref[...] = a1.astype(jnp.int32)
    cnt_ref[...] = jnp.sum(oh0.astype(jnp.int32) + oh1.astype(jnp.int32),
                           axis=0, keepdims=True)

    f_i = (oh0.astype(jnp.float32) + oh1.astype(jnp.float32)).mean(axis=0) / TOPK
    p_i = probs.mean(axis=0)
    aux_ref[...] = (NE * jnp.sum(f_i * p_i)).reshape(1, 1)


def _run_router(x_flat, router_w):
    T = x_flat.shape[0]
    return pl.pallas_call(
        _router_kernel,
        out_shape=(
            jax.ShapeDtypeStruct((T, 1), jnp.int32),
            jax.ShapeDtypeStruct((T, 1), jnp.int32),
            jax.ShapeDtypeStruct((T, 1), jnp.float32),
            jax.ShapeDtypeStruct((T, 1), jnp.float32),
            jax.ShapeDtypeStruct((1, NE), jnp.int32),
            jax.ShapeDtypeStruct((1, 1), jnp.float32),
        ),
        in_specs=[
            pl.BlockSpec((T, DIM), lambda: (0, 0)),
            pl.BlockSpec((NE, DIM), lambda: (0, 0)),
        ],
        out_specs=(
            pl.BlockSpec((T, 1), lambda: (0, 0)),
            pl.BlockSpec((T, 1), lambda: (0, 0)),
            pl.BlockSpec((T, 1), lambda: (0, 0)),
            pl.BlockSpec((T, 1), lambda: (0, 0)),
            pl.BlockSpec((1, NE), lambda: (0, 0)),
            pl.BlockSpec((1, 1), lambda: (0, 0)),
        ),
    )(x_flat, router_w)


def _sc_gather_rows(table, idx):
    n, d = idx.shape[0], table.shape[1]
    info = plsc.get_sparse_core_info()
    nw = info.num_cores * info.num_subcores
    per_w = n // nw
    chunk = next(c for c in (48, 32, 16, 8) if per_w % c == 0)
    nch = per_w // chunk
    mesh = plsc.VectorSubcoreMesh(core_axis_name="c", subcore_axis_name="s")

    @functools.partial(
        pl.kernel, mesh=mesh,
        out_type=jax.ShapeDtypeStruct((n, d), jnp.float32),
        scratch_types=[
            pltpu.VMEM((per_w,), jnp.int32),
            pltpu.VMEM((chunk, d), jnp.float32),
            pltpu.VMEM((chunk, d), jnp.float32),
            pltpu.SemaphoreType.DMA,
            pltpu.SemaphoreType.DMA,
        ],
    )
    def k(table_hbm, idx_hbm, out_hbm, idx_v, buf_a, buf_b, sem_a, sem_b):
        wid = lax.axis_index("s") * info.num_cores + lax.axis_index("c")
        base = wid * per_w
        pltpu.sync_copy(idx_hbm.at[pl.ds(base, per_w)], idx_v)

        bufs = (buf_a, buf_b)
        sems = (sem_a, sem_b)
        inflight = [None, None]
        for c in range(nch):
            sl = idx_v.at[pl.ds(c * chunk, chunk)]
            inflight[c % 2] = pltpu.async_copy(
                table_hbm.at[sl], bufs[c % 2], sems[c % 2])
            if c >= 1:
                inflight[(c - 1) % 2].wait()
                pltpu.sync_copy(bufs[(c - 1) % 2],
                                out_hbm.at[pl.ds(base + (c - 1) * chunk, chunk)])
        inflight[(nch - 1) % 2].wait()
        pltpu.sync_copy(bufs[(nch - 1) % 2],
                        out_hbm.at[pl.ds(base + (nch - 1) * chunk, chunk)])

    return k(table, idx)


def _sum2_kernel(ab_ref, g0_ref, g1_ref, out_ref):
    g0 = g0_ref[...]
    g1 = g1_ref[...]
    out_ref[...] = g0 * ab_ref[0] + g1 * ab_ref[1]


def _run_sum2(ab, g0, g1, t_tok):
    bt = 512
    return pl.pallas_call(
        _sum2_kernel,
        grid=(t_tok // bt,),
        in_specs=[
            pl.BlockSpec((2, bt, DIM), lambda i: (0, i, 0)),
            pl.BlockSpec((bt, 1), lambda i: (i, 0)),
            pl.BlockSpec((bt, 1), lambda i: (i, 0)),
        ],
        out_specs=pl.BlockSpec((bt, DIM), lambda i: (i, 0)),
        out_shape=jax.ShapeDtypeStruct((t_tok, DIM), jnp.float32),
        compiler_params=pltpu.CompilerParams(
            dimension_semantics=("parallel",)),
    )(ab, g0, g1)


def _ffn_kernel(eid_ref, act_ref, x_ref, w1_ref, b1_ref, w2_ref, b2_ref,
                out_ref, acc_ref):
    i = pl.program_id(0)
    j = pl.program_id(1)

    @pl.when(act_ref[i] == 1)
    def _compute():
        x = x_ref[...].astype(jnp.bfloat16)
        h = jnp.dot(x, w1_ref[0].astype(jnp.bfloat16),
                    preferred_element_type=jnp.float32)
        h = h + b1_ref[0, 0][None, :]
        h = _gelu_exact(h)
        y = jnp.dot(h.astype(jnp.bfloat16), w2_ref[0].astype(jnp.bfloat16),
                    preferred_element_type=jnp.float32)

        @pl.when(j == 0)
        def _set():
            acc_ref[...] = y

        @pl.when(j != 0)
        def _acc():
            acc_ref[...] += y

        @pl.when(j == NJ - 1)
        def _emit():
            out_ref[...] = acc_ref[...] + b2_ref[0, 0][None, :]


def _run_ffn(xg, W1, b1, W2, b2, tile_eid, tile_act):
    b1r = b1.reshape(NE, 1, HID)
    b2r = b2.reshape(NE, 1, DIM)
    grid_spec = pltpu.PrefetchScalarGridSpec(
        num_scalar_prefetch=2,
        grid=(NT, NJ),
        in_specs=[
            pl.BlockSpec((BT, DIM), lambda i, j, e_r, a_r: (i, 0)),
            pl.BlockSpec((1, DIM, BH), lambda i, j, e_r, a_r: (e_r[i], 0, j)),
            pl.BlockSpec((1, 1, BH), lambda i, j, e_r, a_r: (e_r[i], 0, j)),
            pl.BlockSpec((1, BH, DIM), lambda i, j, e_r, a_r: (e_r[i], j, 0)),
            pl.BlockSpec((1, 1, DIM), lambda i, j, e_r, a_r: (e_r[i], 0, 0)),
        ],
        out_specs=pl.BlockSpec((BT, DIM), lambda i, j, e_r, a_r: (i, 0)),
        scratch_shapes=[pltpu.VMEM((BT, DIM), jnp.float32)],
    )
    return pl.pallas_call(
        _ffn_kernel,
        grid_spec=grid_spec,
        out_shape=jax.ShapeDtypeStruct((RPAD, DIM), jnp.float32),
    )(tile_eid, tile_act, xg, W1, b1r, W2, b2r)


def kernel(x, router_w, W1, b1, W2, b2):
    b, s, d = x.shape
    T = b * s
    x_flat = x.reshape(T, d)

    a0c, a1c, g0, g1, cntr, aux = _run_router(x_flat, router_w)
    a0f, a1f = a0c[:, 0], a1c[:, 0]
    cnt = cntr[0]

    iota_e = jnp.arange(NE, dtype=jnp.int32)[None, :]
    oh0 = (a0f[:, None] == iota_e).astype(jnp.int32)
    oh1 = (a1f[:, None] == iota_e).astype(jnp.int32)
    c0 = oh0.sum(axis=0)
    r0 = jnp.take_along_axis(jnp.cumsum(oh0, axis=0) - oh0, a0c, axis=1)[:, 0]
    r1 = jnp.take_along_axis(jnp.cumsum(oh1, axis=0) - oh1, a1c, axis=1)[:, 0]
    padded0 = ((cnt + BT - 1) // BT) * BT
    pcum0 = jnp.cumsum(padded0)
    start = pcum0 - padded0
    p0 = start[a0f] + r0
    p1 = start[a1f] + c0[a1f] + r1
    tok = jnp.arange(T, dtype=jnp.int32)
    row_token = (jnp.arange(RPAD, dtype=jnp.int32) % T) \
        .at[jnp.concatenate([p0, p1])].set(jnp.concatenate([tok, tok]))

    padded = ((cnt + BT - 1) // BT) * BT
    pcum = jnp.cumsum(padded)
    total = pcum[-1]
    tile_start = jnp.arange(NT, dtype=jnp.int32) * BT
    tile_eid = jnp.searchsorted(pcum, tile_start, side="right").astype(jnp.int32)
    tile_act = (tile_start < total).astype(jnp.int32)
    n_active = total // BT
    last_eid = tile_eid[jnp.maximum(n_active - 1, 0)]
    tile_eid = jnp.where(tile_act == 1, tile_eid, last_eid)

    xg = _sc_gather_rows(x_flat, row_token)
    yp = _run_ffn(xg, W1, b1, W2, b2, tile_eid, tile_act)
    pids = jnp.concatenate([p0, p1]).astype(jnp.int32)
    ab = _sc_gather_rows(yp, pids).reshape(2, T, d)
    out = _run_sum2(ab, g0, g1, T)

    return out.reshape(b, s, d), aux[0, 0]


# speedup vs baseline: 1.1378x; 1.1378x over previous
import functools

import jax
import jax.numpy as jnp
from jax import lax
from jax.experimental import pallas as pl
from jax.experimental.pallas import tpu as pltpu
from jax.experimental.pallas import tpu_sc as plsc

DIM = 1024
HID = 4096
NE = 8
TOPK = 2

BT = 512
BH = 2048
NJ = HID // BH
T_TOK = 4096
RPAD = TOPK * T_TOK + NE * BT
NT = RPAD // BT

_NEG_INF = float("-inf")
_INV_SQRT2 = 0.7071067811865476


def _gelu_exact(v):
    return 0.5 * v * (1.0 + jax.lax.erf(v * _INV_SQRT2))


def _router_kernel(x_ref, rw_ref, p0_ref, p1_ref, g0_ref, g1_ref, cnt_ref,
                   aux_ref):
    x = x_ref[...]
    rw = rw_ref[...]
    logits = jax.lax.dot_general(
        x, rw, (((1,), (1,)), ((), ())),
        preferred_element_type=jnp.float32)
    m = jnp.max(logits, axis=-1, keepdims=True)
    ex = jnp.exp(logits - m)
    probs = ex / jnp.sum(ex, axis=-1, keepdims=True)

    T = x.shape[0]
    iota_e = jax.lax.broadcasted_iota(jnp.int32, (T, NE), 1)
    a0 = jnp.argmax(probs, axis=-1)[:, None]
    oh0 = (iota_e == a0)
    m0 = jnp.max(probs, axis=-1, keepdims=True)
    masked = jnp.where(oh0, _NEG_INF, probs)
    a1 = jnp.argmax(masked, axis=-1)[:, None]
    oh1 = (iota_e == a1)
    m1 = jnp.max(masked, axis=-1, keepdims=True)
    s = m0 + m1
    g0_ref[...] = m0 / s
    g1_ref[...] = m1 / s

    p0_ref[...] = a0.astype(jnp.int32)
    p1_ref[...] = a1.astype(jnp.int32)
    cnt_ref[...] = jnp.sum(oh0.astype(jnp.int32) + oh1.astype(jnp.int32),
                           axis=0, keepdims=True)

    f_i = (oh0.astype(jnp.float32) + oh1.astype(jnp.float32)).mean(axis=0) / TOPK
    p_i = probs.mean(axis=0)
    aux_ref[...] = (NE * jnp.sum(f_i * p_i)).reshape(1, 1)


def _run_router(x_flat, router_w):
    T = x_flat.shape[0]
    return pl.pallas_call(
        _router_kernel,
        out_shape=(
            jax.ShapeDtypeStruct((T, 1), jnp.int32),
            jax.ShapeDtypeStruct((T, 1), jnp.int32),
            jax.ShapeDtypeStruct((T, 1), jnp.float32),
            jax.ShapeDtypeStruct((T, 1), jnp.float32),
            jax.ShapeDtypeStruct((1, NE), jnp.int32),
            jax.ShapeDtypeStruct((1, 1), jnp.float32),
        ),
        in_specs=[
            pl.BlockSpec((T, DIM), lambda: (0, 0)),
            pl.BlockSpec((NE, DIM), lambda: (0, 0)),
        ],
        out_specs=(
            pl.BlockSpec((T, 1), lambda: (0, 0)),
            pl.BlockSpec((T, 1), lambda: (0, 0)),
            pl.BlockSpec((T, 1), lambda: (0, 0)),
            pl.BlockSpec((T, 1), lambda: (0, 0)),
            pl.BlockSpec((1, NE), lambda: (0, 0)),
            pl.BlockSpec((1, 1), lambda: (0, 0)),
        ),
    )(x_flat, router_w)


def _sc_gather_rows(table, idx):
    n, d = idx.shape[0], table.shape[1]
    info = plsc.get_sparse_core_info()
    nw = info.num_cores * info.num_subcores
    per_w = n // nw
    chunk = 32
    nch = per_w // chunk
    mesh = plsc.VectorSubcoreMesh(core_axis_name="c", subcore_axis_name="s")

    @functools.partial(
        pl.kernel, mesh=mesh,
        out_type=jax.ShapeDtypeStruct((n, d), jnp.float32),
        scratch_types=[
            pltpu.VMEM((per_w,), jnp.int32),
            pltpu.VMEM((chunk, d), jnp.float32),
            pltpu.VMEM((chunk, d), jnp.float32),
            pltpu.SemaphoreType.DMA,
            pltpu.SemaphoreType.DMA,
        ],
    )
    def k(table_hbm, idx_hbm, out_hbm, idx_v, buf_a, buf_b, sem_a, sem_b):
        wid = lax.axis_index("s") * info.num_cores + lax.axis_index("c")
        base = wid * per_w
        pltpu.sync_copy(idx_hbm.at[pl.ds(base, per_w)], idx_v)

        bufs = (buf_a, buf_b)
        sems = (sem_a, sem_b)
        inflight = [None, None]
        for c in range(nch):
            sl = idx_v.at[pl.ds(c * chunk, chunk)]
            inflight[c % 2] = pltpu.async_copy(
                table_hbm.at[sl], bufs[c % 2], sems[c % 2])
            if c >= 1:
                inflight[(c - 1) % 2].wait()
                pltpu.sync_copy(bufs[(c - 1) % 2],
                                out_hbm.at[pl.ds(base + (c - 1) * chunk, chunk)])
        inflight[(nch - 1) % 2].wait()
        pltpu.sync_copy(bufs[(nch - 1) % 2],
                        out_hbm.at[pl.ds(base + (nch - 1) * chunk, chunk)])

    return k(table, idx)


def _sc_scatter_rows(table, i0, i1, n_out):
    t_tok, d = table.shape
    info = plsc.get_sparse_core_info()
    nw = info.num_cores * info.num_subcores
    per_w = t_tok // nw
    chunk = 32
    nch = per_w // chunk
    mesh = plsc.VectorSubcoreMesh(core_axis_name="c", subcore_axis_name="s")

    @functools.partial(
        pl.kernel, mesh=mesh,
        out_type=jax.ShapeDtypeStruct((n_out, d), jnp.float32),
        scratch_types=[
            pltpu.VMEM((nch, chunk), jnp.int32),
            pltpu.VMEM((nch, chunk), jnp.int32),
            pltpu.VMEM((chunk, d), jnp.float32),
            pltpu.VMEM((chunk, d), jnp.float32),
            pltpu.SemaphoreType.DMA,
            pltpu.SemaphoreType.DMA,
            pltpu.SemaphoreType.DMA,
            pltpu.SemaphoreType.DMA,
        ],
    )
    def k(table_hbm, i0_hbm, i1_hbm, out_hbm, i0_v, i1_v, buf_a, buf_b,
          s0a, s1a, s0b, s1b):
        wid = lax.axis_index("s") * info.num_cores + lax.axis_index("c")
        base = wid * per_w
        pltpu.sync_copy(i0_hbm.at[wid], i0_v)
        pltpu.sync_copy(i1_hbm.at[wid], i1_v)

        bufs = (buf_a, buf_b)
        sems = ((s0a, s1a), (s0b, s1b))
        pend = [None, None]
        for c in range(nch):
            r = c % 2
            if pend[r] is not None:
                pend[r][0].wait()
                pend[r][1].wait()
            pltpu.sync_copy(table_hbm.at[pl.ds(base + c * chunk, chunk)],
                            bufs[r])
            w0 = pltpu.async_copy(bufs[r], out_hbm.at[i0_v.at[c]], sems[r][0])
            w1 = pltpu.async_copy(bufs[r], out_hbm.at[i1_v.at[c]], sems[r][1])
            pend[r] = (w0, w1)
        for r in range(2):
            if pend[r] is not None:
                pend[r][0].wait()
                pend[r][1].wait()

    return k(table, i0, i1)


def _sum2_kernel(ab_ref, g0_ref, g1_ref, out_ref):
    g0 = g0_ref[...]
    g1 = g1_ref[...]
    out_ref[...] = g0 * ab_ref[0] + g1 * ab_ref[1]


def _run_sum2(ab, g0, g1, t_tok):
    bt = 512
    return pl.pallas_call(
        _sum2_kernel,
        grid=(t_tok // bt,),
        in_specs=[
            pl.BlockSpec((2, bt, DIM), lambda i: (0, i, 0)),
            pl.BlockSpec((bt, 1), lambda i: (i, 0)),
            pl.BlockSpec((bt, 1), lambda i: (i, 0)),
        ],
        out_specs=pl.BlockSpec((bt, DIM), lambda i: (i, 0)),
        out_shape=jax.ShapeDtypeStruct((t_tok, DIM), jnp.float32),
        compiler_params=pltpu.CompilerParams(
            dimension_semantics=("parallel",)),
    )(ab, g0, g1)


def _ffn_kernel(eid_ref, act_ref, x_ref, w1_ref, b1_ref, w2_ref, b2_ref,
                out_ref, acc_ref):
    i = pl.program_id(0)
    j = pl.program_id(1)

    @pl.when(act_ref[i] == 1)
    def _compute():
        x = x_ref[...].astype(jnp.bfloat16)
        h = jnp.dot(x, w1_ref[0].astype(jnp.bfloat16),
                    preferred_element_type=jnp.float32)
        h = h + b1_ref[0, 0][None, :]
        h = _gelu_exact(h)
        y = jnp.dot(h.astype(jnp.bfloat16), w2_ref[0].astype(jnp.bfloat16),
                    preferred_element_type=jnp.float32)

        @pl.when(j == 0)
        def _set():
            acc_ref[...] = y

        @pl.when(j != 0)
        def _acc():
            acc_ref[...] += y

        @pl.when(j == NJ - 1)
        def _emit():
            out_ref[...] = acc_ref[...] + b2_ref[0, 0][None, :]


def _run_ffn(xg, W1, b1, W2, b2, tile_eid, tile_act):
    b1r = b1.reshape(NE, 1, HID)
    b2r = b2.reshape(NE, 1, DIM)
    grid_spec = pltpu.PrefetchScalarGridSpec(
        num_scalar_prefetch=2,
        grid=(NT, NJ),
        in_specs=[
            pl.BlockSpec((BT, DIM), lambda i, j, e_r, a_r: (i, 0)),
            pl.BlockSpec((1, DIM, BH), lambda i, j, e_r, a_r: (e_r[i], 0, j)),
            pl.BlockSpec((1, 1, BH), lambda i, j, e_r, a_r: (e_r[i], 0, j)),
            pl.BlockSpec((1, BH, DIM), lambda i, j, e_r, a_r: (e_r[i], j, 0)),
            pl.BlockSpec((1, 1, DIM), lambda i, j, e_r, a_r: (e_r[i], 0, 0)),
        ],
        out_specs=pl.BlockSpec((BT, DIM), lambda i, j, e_r, a_r: (i, 0)),
        scratch_shapes=[pltpu.VMEM((BT, DIM), jnp.float32)],
    )
    return pl.pallas_call(
        _ffn_kernel,
        grid_spec=grid_spec,
        out_shape=jax.ShapeDtypeStruct((RPAD, DIM), jnp.float32),
    )(tile_eid, tile_act, xg, W1, b1r, W2, b2r)


def kernel(x, router_w, W1, b1, W2, b2):
    b, s, d = x.shape
    T = b * s
    x_flat = x.reshape(T, d)

    a0c, a1c, g0, g1, cntr, aux = _run_router(x_flat, router_w)
    a0f, a1f = a0c[:, 0], a1c[:, 0]
    cnt = cntr[0]

    iota_e = jnp.arange(NE, dtype=jnp.int32)[None, :]
    oh0 = (a0f[:, None] == iota_e).astype(jnp.int32)
    oh1 = (a1f[:, None] == iota_e).astype(jnp.int32)
    c0 = oh0.sum(axis=0)
    r0 = jnp.take_along_axis(jnp.cumsum(oh0, axis=0) - oh0, a0c, axis=1)[:, 0]
    r1 = jnp.take_along_axis(jnp.cumsum(oh1, axis=0) - oh1, a1c, axis=1)[:, 0]
    padded0 = ((cnt + BT - 1) // BT) * BT
    pcum0 = jnp.cumsum(padded0)
    start = pcum0 - padded0
    p0 = start[a0f] + r0
    p1 = start[a1f] + c0[a1f] + r1
    padded = ((cnt + BT - 1) // BT) * BT
    pcum = jnp.cumsum(padded)
    total = pcum[-1]
    tile_start = jnp.arange(NT, dtype=jnp.int32) * BT
    tile_eid = jnp.searchsorted(pcum, tile_start, side="right").astype(jnp.int32)
    tile_act = (tile_start < total).astype(jnp.int32)
    n_active = total // BT
    last_eid = tile_eid[jnp.maximum(n_active - 1, 0)]
    tile_eid = jnp.where(tile_act == 1, tile_eid, last_eid)

    nw_sc = 32
    i0_3 = p0.astype(jnp.int32).reshape(nw_sc, -1, 32)
    i1_3 = p1.astype(jnp.int32).reshape(nw_sc, -1, 32)
    xg = _sc_scatter_rows(x_flat, i0_3, i1_3, RPAD)
    yp = _run_ffn(xg, W1, b1, W2, b2, tile_eid, tile_act)
    pids = jnp.concatenate([p0, p1]).astype(jnp.int32)
    ab = _sc_gather_rows(yp, pids).reshape(2, T, d)
    out = _run_sum2(ab, g0, g1, T)

    return out.reshape(b, s, d), aux[0, 0]
